# baseline (speedup 1.0000x reference)
import functools

import jax
import jax.numpy as jnp
from jax.experimental import pallas as pl
from jax.experimental.pallas import tpu as pltpu

_EMB_DIM = 300


_N_SEMS = 8


def _gather_kernel(idx_ref, table_ref, out_ref, scratch_ref, sems, *,
                   rows, v_max):
    base = pl.program_id(0) * rows
    for r in range(rows):
        row = jnp.minimum(jnp.maximum(idx_ref[base + r], 0), v_max)
        pltpu.make_async_copy(table_ref.at[pl.ds(row, 1)],
                              scratch_ref.at[pl.ds(r, 1)],
                              sems.at[r % _N_SEMS]).start(priority=r % 2)
    for k in range(_N_SEMS):
        pltpu.make_async_copy(table_ref.at[pl.ds(0, rows // _N_SEMS)],
                              scratch_ref.at[pl.ds(0, rows // _N_SEMS)],
                              sems.at[k]).wait()
    out_ref[...] = scratch_ref[:, :_EMB_DIM]


def kernel(table_padded, indices):
    v_pad, d_pad = table_padded.shape
    seq = int(indices.shape[0])

    rows = 2048
    while seq % rows:
        rows //= 2
    n_steps = seq // rows

    idx = indices.astype(jnp.int32)
    out = pl.pallas_call(
        functools.partial(_gather_kernel, rows=rows, v_max=v_pad - 1),
        out_shape=jax.ShapeDtypeStruct((seq, _EMB_DIM), table_padded.dtype),
        grid_spec=pltpu.PrefetchScalarGridSpec(
            num_scalar_prefetch=1,
            grid=(n_steps,),
            in_specs=[pl.BlockSpec(memory_space=pl.ANY)],
            out_specs=pl.BlockSpec((rows, _EMB_DIM), lambda i, ix: (i, 0)),
            scratch_shapes=[pltpu.VMEM((rows, d_pad), table_padded.dtype),
                            pltpu.SemaphoreType.DMA((_N_SEMS,))],
        ),
        compiler_params=pltpu.CompilerParams(
            dimension_semantics=("parallel",),
            disable_bounds_checks=True),
    )(idx, table_padded)
    return out


# speedup vs baseline: 1.0015x; 1.0015x over previous
import functools

import jax
import jax.numpy as jnp
from jax.experimental import pallas as pl
from jax.experimental.pallas import tpu as pltpu

_EMB_DIM = 300


def _gather_kernel(idx_ref, table_ref, out_ref, scratch_ref, sem, *,
                   rows, v_max):
    base = pl.program_id(0) * rows
    for r in range(rows):
        row = jnp.minimum(jnp.maximum(idx_ref[base + r], 0), v_max)
        pltpu.make_async_copy(table_ref.at[pl.ds(row, 1)],
                              scratch_ref.at[pl.ds(r, 1)],
                              sem).start(priority=r % 2)
    pltpu.make_async_copy(table_ref.at[pl.ds(0, rows)], scratch_ref,
                          sem).wait()
    out_ref[...] = scratch_ref[:, :_EMB_DIM]


def kernel(table_padded, indices):
    v_pad, d_pad = table_padded.shape
    seq = int(indices.shape[0])

    rows = 2048
    while seq % rows:
        rows //= 2
    n_steps = seq // rows

    idx = indices.astype(jnp.int32)
    out = pl.pallas_call(
        functools.partial(_gather_kernel, rows=rows, v_max=v_pad - 1),
        out_shape=jax.ShapeDtypeStruct((seq, _EMB_DIM), table_padded.dtype),
        grid_spec=pltpu.PrefetchScalarGridSpec(
            num_scalar_prefetch=1,
            grid=(n_steps,),
            in_specs=[pl.BlockSpec(memory_space=pl.ANY)],
            out_specs=pl.BlockSpec((rows, _EMB_DIM), lambda i, ix: (i, 0)),
            scratch_shapes=[pltpu.VMEM((rows, d_pad), table_padded.dtype),
                            pltpu.SemaphoreType.DMA],
        ),
        compiler_params=pltpu.CompilerParams(
            dimension_semantics=("parallel",),
            disable_bounds_checks=True),
    )(idx, table_padded)
    return out
